# baseline (speedup 1.0000x reference)
import functools

import jax
import jax.numpy as jnp
from jax import lax
from jax.experimental import pallas as pl
from jax.experimental.pallas import tpu as pltpu


def _det_cls_kernel(x_ref, cnt0_ref, len_ref,
                    wemb_ref, wrest_ref, w0_ref, wcc0_ref, b0_ref,
                    wihf_ref, bihf_ref, whhf_ref, bhnf_ref,
                    wihb_ref, bihb_ref, bhnb_ref,
                    w1f_ref, w1b_ref, b1_ref, w2_ref, b2_ref, w3_ref, b3_ref,
                    out_ref,
                    h_scr, xl_scr,
                    *, S, Sc, Bb, H, F, D, L, pad_label, cc_label, ln_eps):
    f32 = jnp.float32
    j = pl.program_id(1)
    nj = pl.num_programs(1)
    lens = len_ref[...]
    lab_iota = lax.broadcasted_iota(jnp.int32, (L - 2, 1), 0)

    wemb = wemb_ref[...]
    wrest = wrest_ref[...]
    wihf = wihf_ref[...]
    bihf = bihf_ref[...]
    whh = whhf_ref[...]
    bhn = bhnf_ref[...]

    t0 = j * Sc

    def body(s, carry):
        x_last, h = carry
        t = t0 + s
        xs = x_ref[s]
        labels = xs[0:1, :].astype(jnp.int32)
        onehot = (labels == lab_iota).astype(f32)
        fe = (jnp.dot(wemb, onehot, preferred_element_type=f32)
              + jnp.dot(wrest, xs[1:6, :], preferred_element_type=f32))
        mu = fe[F:F + 1]
        d = fe[0:F] - mu
        var = jnp.mean(d * d, axis=0, keepdims=True)
        feat = d * lax.rsqrt(var + ln_eps)
        gi = jnp.dot(wihf, feat, preferred_element_type=f32) + bihf

        gh = jnp.dot(whh, h, preferred_element_type=f32)
        r = jax.nn.sigmoid(gi[0:H] + gh[0:H])
        z = jax.nn.sigmoid(gi[H:2 * H] + gh[H:2 * H])
        n = jnp.tanh(gi[2 * H:3 * H] + r * (gh[2 * H:3 * H] + bhn))
        h_new = (1.0 - z) * n + z * h
        h = jnp.where(t < lens, h_new, h)
        x_last = jnp.where(t == lens - 1, feat, x_last)
        return x_last, h

    @pl.when(j == 0)
    def _first_chunk():
        xs0 = x_ref[0]
        fe0 = (jnp.dot(w0_ref[...], xs0[1:, :], preferred_element_type=f32)
               + jnp.dot(wcc0_ref[...], cnt0_ref[...],
                         preferred_element_type=f32)
               + b0_ref[...])
        mu0 = fe0[F:F + 1]
        d0 = fe0[0:F] - mu0
        var0 = jnp.mean(d0 * d0, axis=0, keepdims=True)
        feat0 = d0 * lax.rsqrt(var0 + ln_eps)
        gi0 = jnp.dot(wihf, feat0, preferred_element_type=f32) + bihf
        r0 = jax.nn.sigmoid(gi0[0:H])
        z0 = jax.nn.sigmoid(gi0[H:2 * H])
        n0 = jnp.tanh(gi0[2 * H:3 * H] + r0 * bhn)
        h1 = (1.0 - z0) * n0
        x_last, h_fwd = lax.fori_loop(
            1, Sc, body,
            (jnp.where(lens == 1, feat0, 0.0), h1), unroll=8)
        h_scr[...] = h_fwd
        xl_scr[...] = x_last

    @pl.when(j > 0)
    def _later_chunk():
        x_last, h_fwd = lax.fori_loop(
            0, Sc, body, (xl_scr[...], h_scr[...]), unroll=8)
        h_scr[...] = h_fwd
        xl_scr[...] = x_last

    @pl.when(j == nj - 1)
    def _finish():
        h_fwd = h_scr[...]
        x_last = xl_scr[...]
        gib = (jnp.dot(wihb_ref[...], x_last, preferred_element_type=f32)
               + bihb_ref[...])
        r_b = jax.nn.sigmoid(gib[0:H])
        z_b = jax.nn.sigmoid(gib[H:2 * H])
        n_b = jnp.tanh(gib[2 * H:3 * H] + r_b * bhnb_ref[...])
        h_bwd = (1.0 - z_b) * n_b

        h1 = jnp.maximum(
            jnp.dot(w1f_ref[...], h_fwd, preferred_element_type=f32)
            + jnp.dot(w1b_ref[...], h_bwd, preferred_element_type=f32)
            + b1_ref[...], 0.0)
        h2 = jnp.maximum(
            jnp.dot(w2_ref[...], h1, preferred_element_type=f32)
            + b2_ref[...], 0.0)
        logit = jnp.sum(w3_ref[...] * h2, axis=0, keepdims=True) + b3_ref[...]
        out_ref[...] = jax.nn.sigmoid(logit)


def kernel(X, lengths, emb, w_cc, b_cc, gamma, beta,
           w_ih_f, w_hh_f, b_ih_f, b_hh_f, w_ih_b, w_hh_b, b_ih_b, b_hh_b,
           w1, b1, w2, b2, w3, b3):
    PAD_LABEL, CC_LABEL = 9, 8
    B, S, D = X.shape
    L, E = emb.shape
    C, CCD = w_cc.shape
    H = w_hh_f.shape[0]
    F = E + 5 + CCD
    Bb = 4096
    Sc = next(c for c in range(min(64, S), 0, -1) if S % c == 0)

    wemb_t = jnp.zeros((F, L), jnp.float32).at[:E, :].set(emb.T)
    wrest_t = (jnp.zeros((F, D - 1), jnp.float32)
               .at[E:E + 5, :5].set(jnp.eye(5, dtype=jnp.float32))
               .at[E + 5:, 5:].set(w_cc.T))
    bcc_t = jnp.zeros((F, 1), jnp.float32).at[E + 5:, :].set(b_cc.T)
    wemb_t = jnp.concatenate([wemb_t, wemb_t.sum(0, keepdims=True) / F], 0)
    wrest_t = jnp.concatenate([wrest_t, wrest_t.sum(0, keepdims=True) / F], 0)
    bcc_t = jnp.concatenate([bcc_t, bcc_t.sum(0, keepdims=True) / F], 0)
    gamma_row = gamma.reshape(1, F)
    beta_col = beta.reshape(F, 1)

    def fuse_bias(b_ih, b_hh):
        return jnp.concatenate(
            [b_ih[:, :2 * H] + b_hh[:, :2 * H], b_ih[:, 2 * H:]], axis=1).T

    wihf_t = w_ih_f.T * gamma_row
    whhf_t = w_hh_f.T
    bihf_t = fuse_bias(b_ih_f, b_hh_f) + w_ih_f.T @ beta_col
    bhnf_t = b_hh_f[:, 2 * H:].T
    wihb_t = w_ih_b.T * gamma_row
    bihb_t = fuse_bias(b_ih_b, b_hh_b) + w_ih_b.T @ beta_col
    bhnb_t = b_hh_b[:, 2 * H:].T

    w1f_t = w1[:H, :].T
    w1b_t = w1[H:, :].T
    b1_t = b1.T
    w2_t = w2.T
    b2_t = b2.T
    w3_t = w3
    b3_t = b3

    lengths = jnp.clip(lengths.astype(jnp.int32).reshape(B), 1, S)
    B_pad = ((B + Bb - 1) // Bb) * Bb
    X = X.astype(jnp.float32)
    if B_pad != B:
        x_fill = jnp.zeros((B_pad - B, S, D), jnp.float32).at[:, :, 0].set(
            float(PAD_LABEL))
        X = jnp.concatenate([X, x_fill], axis=0)
        lengths = jnp.concatenate(
            [lengths, jnp.ones((B_pad - B,), jnp.int32)], axis=0)

    X3 = jnp.transpose(X[:, :, :6], (1, 2, 0))
    cnt0 = X[:, 0, 6:].T
    len2 = lengths.reshape(1, B_pad)

    wemb_s = wemb_t[:, :L - 2]
    wrest_s = wrest_t[:, :5]
    w0_t = wrest_t[:, :5]
    wcc0_t = wrest_t[:, 5:]
    b0_t = wemb_t[:, CC_LABEL:CC_LABEL + 1] + bcc_t
    weights = (wemb_s, wrest_s, w0_t, wcc0_t, b0_t,
               wihf_t, bihf_t, whhf_t, bhnf_t,
               wihb_t, bihb_t, bhnb_t,
               w1f_t, w1b_t, b1_t, w2_t, b2_t, w3_t, b3_t)

    kern = functools.partial(_det_cls_kernel, S=S, Sc=Sc, Bb=Bb, H=H, F=F,
                             D=D, L=L, pad_label=PAD_LABEL,
                             cc_label=CC_LABEL, ln_eps=1e-5)

    def full2d(arr):
        return pl.BlockSpec(arr.shape, lambda i, j: (0, 0))

    out = pl.pallas_call(
        kern,
        out_shape=jax.ShapeDtypeStruct((1, B_pad), jnp.float32),
        grid=(B_pad // Bb, S // Sc),
        in_specs=[pl.BlockSpec((Sc, 6, Bb), lambda i, j: (j, 0, i)),
                  pl.BlockSpec((C, Bb), lambda i, j: (0, i)),
                  pl.BlockSpec((1, Bb), lambda i, j: (0, i))]
                 + [full2d(w) for w in weights],
        out_specs=pl.BlockSpec((1, Bb), lambda i, j: (0, i)),
        scratch_shapes=[pltpu.VMEM((H, Bb), jnp.float32),
                        pltpu.VMEM((F, Bb), jnp.float32)],
        compiler_params=pltpu.CompilerParams(
            dimension_semantics=("parallel", "arbitrary"),
            vmem_limit_bytes=64 * 1024 * 1024),
    )(X3, cnt0, len2, *weights)
    return out[0, :B]


# speedup vs baseline: 1.2167x; 1.2167x over previous
import functools

import jax
import jax.numpy as jnp
from jax import lax
from jax.experimental import pallas as pl
from jax.experimental.pallas import tpu as pltpu


def _det_cls_kernel(x_ref, len_ref,
                    wemb_ref, wrest_ref, w0_ref, b0_ref,
                    wihf_ref, bihf_ref, whhf_ref, bhnf_ref,
                    wihb_ref, bihb_ref, bhnb_ref,
                    w1f_ref, w1b_ref, b1_ref, w2_ref, b2_ref, w3_ref, b3_ref,
                    out_ref,
                    h_scr, xl_scr,
                    *, S, Sc, Bb, H, F, D, L, pad_label, cc_label, ln_eps):
    f32 = jnp.float32
    j = pl.program_id(1)
    nj = pl.num_programs(1)
    lens = len_ref[...]
    lab_iota = lax.broadcasted_iota(jnp.int32, (L - 2, 1), 0)

    wemb = wemb_ref[...]
    wrest = wrest_ref[...]
    wihf = wihf_ref[...]
    bihf = bihf_ref[...]
    whh = whhf_ref[...]
    bhn = bhnf_ref[...]

    t0 = j * Sc

    def body(s, carry):
        x_last, h = carry
        t = t0 + s
        xs = x_ref[s]
        labels = xs[0:1, :].astype(jnp.int32)
        onehot = (labels == lab_iota).astype(f32)
        fe = (jnp.dot(wemb, onehot, preferred_element_type=f32)
              + jnp.dot(wrest, xs[1:6, :], preferred_element_type=f32))
        mu = fe[F:F + 1]
        d = fe[0:F] - mu
        var = jnp.mean(d * d, axis=0, keepdims=True)
        feat = d * lax.rsqrt(var + ln_eps)
        gi = jnp.dot(wihf, feat, preferred_element_type=f32) + bihf

        gh = jnp.dot(whh, h, preferred_element_type=f32)
        r = jax.nn.sigmoid(gi[0:H] + gh[0:H])
        z = jax.nn.sigmoid(gi[H:2 * H] + gh[H:2 * H])
        n = jnp.tanh(gi[2 * H:3 * H] + r * (gh[2 * H:3 * H] + bhn))
        h_new = (1.0 - z) * n + z * h
        h = jnp.where(t < lens, h_new, h)
        x_last = jnp.where(t == lens - 1, feat, x_last)
        return x_last, h

    @pl.when(j == 0)
    def _first_chunk():
        xs0 = x_ref[0]
        fe0 = (jnp.dot(w0_ref[...], xs0[1:, :], preferred_element_type=f32)
               + b0_ref[...])
        mu0 = fe0[F:F + 1]
        d0 = fe0[0:F] - mu0
        var0 = jnp.mean(d0 * d0, axis=0, keepdims=True)
        feat0 = d0 * lax.rsqrt(var0 + ln_eps)
        gi0 = jnp.dot(wihf, feat0, preferred_element_type=f32) + bihf
        r0 = jax.nn.sigmoid(gi0[0:H])
        z0 = jax.nn.sigmoid(gi0[H:2 * H])
        n0 = jnp.tanh(gi0[2 * H:3 * H] + r0 * bhn)
        h1 = (1.0 - z0) * n0
        x_last, h_fwd = lax.fori_loop(
            1, Sc, body,
            (jnp.where(lens == 1, feat0, 0.0), h1), unroll=16)
        h_scr[...] = h_fwd
        xl_scr[...] = x_last

    @pl.when(j > 0)
    def _later_chunk():
        x_last, h_fwd = lax.fori_loop(
            0, Sc, body, (xl_scr[...], h_scr[...]), unroll=16)
        h_scr[...] = h_fwd
        xl_scr[...] = x_last

    @pl.when(j == nj - 1)
    def _finish():
        h_fwd = h_scr[...]
        x_last = xl_scr[...]
        gib = (jnp.dot(wihb_ref[...], x_last, preferred_element_type=f32)
               + bihb_ref[...])
        r_b = jax.nn.sigmoid(gib[0:H])
        z_b = jax.nn.sigmoid(gib[H:2 * H])
        n_b = jnp.tanh(gib[2 * H:3 * H] + r_b * bhnb_ref[...])
        h_bwd = (1.0 - z_b) * n_b

        h1 = jnp.maximum(
            jnp.dot(w1f_ref[...], h_fwd, preferred_element_type=f32)
            + jnp.dot(w1b_ref[...], h_bwd, preferred_element_type=f32)
            + b1_ref[...], 0.0)
        h2 = jnp.maximum(
            jnp.dot(w2_ref[...], h1, preferred_element_type=f32)
            + b2_ref[...], 0.0)
        logit = jnp.sum(w3_ref[...] * h2, axis=0, keepdims=True) + b3_ref[...]
        out_ref[...] = jax.nn.sigmoid(logit)


def kernel(X, lengths, emb, w_cc, b_cc, gamma, beta,
           w_ih_f, w_hh_f, b_ih_f, b_hh_f, w_ih_b, w_hh_b, b_ih_b, b_hh_b,
           w1, b1, w2, b2, w3, b3):
    PAD_LABEL, CC_LABEL = 9, 8
    B, S, D = X.shape
    L, E = emb.shape
    C, CCD = w_cc.shape
    H = w_hh_f.shape[0]
    F = E + 5 + CCD
    Bb = 4096
    Sc = next(c for c in range(min(64, S), 0, -1) if S % c == 0)

    wemb_t = jnp.zeros((F, L), jnp.float32).at[:E, :].set(emb.T)
    wrest_t = (jnp.zeros((F, D - 1), jnp.float32)
               .at[E:E + 5, :5].set(jnp.eye(5, dtype=jnp.float32))
               .at[E + 5:, 5:].set(w_cc.T))
    bcc_t = jnp.zeros((F, 1), jnp.float32).at[E + 5:, :].set(b_cc.T)
    wemb_t = jnp.concatenate([wemb_t, wemb_t.sum(0, keepdims=True) / F], 0)
    wrest_t = jnp.concatenate([wrest_t, wrest_t.sum(0, keepdims=True) / F], 0)
    bcc_t = jnp.concatenate([bcc_t, bcc_t.sum(0, keepdims=True) / F], 0)
    gamma_row = gamma.reshape(1, F)
    beta_col = beta.reshape(F, 1)

    def fuse_bias(b_ih, b_hh):
        return jnp.concatenate(
            [b_ih[:, :2 * H] + b_hh[:, :2 * H], b_ih[:, 2 * H:]], axis=1).T

    wihf_t = w_ih_f.T * gamma_row
    whhf_t = w_hh_f.T
    bihf_t = fuse_bias(b_ih_f, b_hh_f) + w_ih_f.T @ beta_col
    bhnf_t = b_hh_f[:, 2 * H:].T
    wihb_t = w_ih_b.T * gamma_row
    bihb_t = fuse_bias(b_ih_b, b_hh_b) + w_ih_b.T @ beta_col
    bhnb_t = b_hh_b[:, 2 * H:].T

    w1f_t = w1[:H, :].T
    w1b_t = w1[H:, :].T
    b1_t = b1.T
    w2_t = w2.T
    b2_t = b2.T
    w3_t = w3
    b3_t = b3

    lengths = jnp.clip(lengths.astype(jnp.int32).reshape(B), 1, S)
    B_pad = ((B + Bb - 1) // Bb) * Bb
    X = X.astype(jnp.float32)
    if B_pad != B:
        x_fill = jnp.zeros((B_pad - B, S, D), jnp.float32).at[:, :, 0].set(
            float(PAD_LABEL))
        X = jnp.concatenate([X, x_fill], axis=0)
        lengths = jnp.concatenate(
            [lengths, jnp.ones((B_pad - B,), jnp.int32)], axis=0)

    X3 = jnp.transpose(X, (1, 2, 0))
    len2 = lengths.reshape(1, B_pad)

    wemb_s = wemb_t[:, :L - 2]
    wrest_s = wrest_t[:, :5]
    w0_t = wrest_t
    b0_t = wemb_t[:, CC_LABEL:CC_LABEL + 1] + bcc_t
    weights = (wemb_s, wrest_s, w0_t, b0_t,
               wihf_t, bihf_t, whhf_t, bhnf_t,
               wihb_t, bihb_t, bhnb_t,
               w1f_t, w1b_t, b1_t, w2_t, b2_t, w3_t, b3_t)

    kern = functools.partial(_det_cls_kernel, S=S, Sc=Sc, Bb=Bb, H=H, F=F,
                             D=D, L=L, pad_label=PAD_LABEL,
                             cc_label=CC_LABEL, ln_eps=1e-5)

    def full2d(arr):
        return pl.BlockSpec(arr.shape, lambda i, j: (0, 0))

    out = pl.pallas_call(
        kern,
        out_shape=jax.ShapeDtypeStruct((1, B_pad), jnp.float32),
        grid=(B_pad // Bb, S // Sc),
        in_specs=[pl.BlockSpec((Sc, D, Bb), lambda i, j: (j, 0, i)),
                  pl.BlockSpec((1, Bb), lambda i, j: (0, i))]
                 + [full2d(w) for w in weights],
        out_specs=pl.BlockSpec((1, Bb), lambda i, j: (0, i)),
        scratch_shapes=[pltpu.VMEM((H, Bb), jnp.float32),
                        pltpu.VMEM((F, Bb), jnp.float32)],
        compiler_params=pltpu.CompilerParams(
            dimension_semantics=("parallel", "arbitrary"),
            vmem_limit_bytes=64 * 1024 * 1024),
    )(X3, len2, *weights)
    return out[0, :B]
